# baseline (speedup 1.0000x reference)
import functools

import jax
import jax.numpy as jnp
from jax.experimental import pallas as pl
from jax.experimental.pallas import tpu as pltpu

_H1 = 0.5
_H2 = 0.5


def _blk(n, target):
    for b in range(min(target, n), 7, -1):
        if n % b == 0 and b % 8 == 0:
            return b
    return n


def _mega_kernel(nf, bmf, nm, bm,
                 s1_ref, s2_ref, w1_ref, w2_ref, b1_ref, b2_ref,
                 adja_ref, adjb_ref, av1_ref, av2_ref, msk_ref, wd_ref,
                 o1_ref, o2_ref,
                 f_sc, h1_sc, h2_sc):
    t = pl.program_id(0)
    nh = av1_ref.shape[1]

    @pl.when(t < nf)
    def _():
        s1 = s1_ref[...]
        s2 = s2_ref[...]
        w1 = w1_ref[...]
        w2 = w2_ref[...]
        nhh = w1_ref.shape[1]
        row = t * bmf
        f_sc[pl.ds(row, bmf), 0 * nhh:1 * nhh] = (
            jnp.dot(s1, w1, preferred_element_type=jnp.float32)
            + b1_ref[...]).astype(jnp.bfloat16)
        f_sc[pl.ds(row, bmf), 1 * nhh:2 * nhh] = (
            jnp.dot(s2, w1, preferred_element_type=jnp.float32)
            + b1_ref[...]).astype(jnp.bfloat16)
        f_sc[pl.ds(row, bmf), 2 * nhh:3 * nhh] = (
            jnp.dot(s1, w2, preferred_element_type=jnp.float32)
            + b2_ref[...]).astype(jnp.bfloat16)
        f_sc[pl.ds(row, bmf), 3 * nhh:4 * nhh] = (
            jnp.dot(s2, w2, preferred_element_type=jnp.float32)
            + b2_ref[...]).astype(jnp.bfloat16)

    @pl.when((t >= nf) & (t < nf + nm))
    def _():
        a = jnp.concatenate(
            [adja_ref[...], adjb_ref[...]], axis=0).astype(jnp.bfloat16)
        g = jax.lax.dot_general(
            a, f_sc[...], (((1,), (0,)), ((), ())),
            preferred_element_type=jnp.float32)
        a1 = av1_ref[...]
        a2 = av2_ref[...]
        g11 = g[:, 0 * nh:1 * nh]
        g21 = g[:, 1 * nh:2 * nh]
        g12 = g[:, 2 * nh:3 * nh]
        g22 = g[:, 3 * nh:4 * nh]
        p11 = jnp.where(g11 > 0, g11, a1 * g11)
        p21 = jnp.where(g21 > 0, g21, a1 * g21)
        p12 = jnp.where(g12 > 0, g12, a2 * g12)
        p22 = jnp.where(g22 > 0, g22, a2 * g22)
        h1v = p11 + _H2 * p22
        h2v = p21 + _H1 * p12
        row = (t - nf) * bm
        h1_sc[pl.ds(row, bm), :] = h1v.astype(jnp.bfloat16)
        h2_sc[pl.ds(row, bm), :] = h2v.astype(jnp.bfloat16)

    @pl.when(t == nf + nm)
    def _():
        m16 = msk_ref[...].astype(jnp.bfloat16)
        craw = jax.lax.dot_general(
            m16, h1_sc[...], (((1,), (0,)), ((), ())),
            preferred_element_type=jnp.float32)
        c = jax.nn.sigmoid(craw / jnp.sum(msk_ref[...]))
        v = jax.lax.dot_general(
            c, wd_ref[...], (((1,), (1,)), ((), ())),
            preferred_element_type=jnp.float32).astype(jnp.bfloat16)
        o1_ref[...] = jax.lax.dot_general(
            v, h1_sc[...], (((1,), (1,)), ((), ())),
            preferred_element_type=jnp.float32)
        o2_ref[...] = jax.lax.dot_general(
            v, h2_sc[...], (((1,), (1,)), ((), ())),
            preferred_element_type=jnp.float32)


def kernel(seq1, seq2, adj, sparse, training, msk, samp_bias1, samp_bias2,
           W1, b1, a1, W2, b2, a2, Wd, bd):
    n = seq1.shape[1]
    d = seq1.shape[2]
    nh = W1.shape[1]
    s1 = seq1[0]
    s2 = seq2[0]
    A = adj[0]

    bmf = n
    nf = 1
    bm = _blk(n, 400)
    nm = n // bm
    av1 = jnp.full((1, nh), a1, jnp.float32)
    av2 = jnp.full((1, nh), a2, jnp.float32)

    def _fidx(t):
        return (0, 0)

    def _aidx(t):
        return (2 * jnp.minimum(jnp.maximum(t - nf, 0), nm - 1), 0)

    def _bidx(t):
        return (2 * jnp.minimum(jnp.maximum(t - nf, 0), nm - 1) + 1, 0)

    o1, o2 = pl.pallas_call(
        functools.partial(_mega_kernel, nf, bmf, nm, bm),
        grid=(nf + nm + 1,),
        in_specs=[
            pl.BlockSpec((bmf, d), _fidx),
            pl.BlockSpec((bmf, d), _fidx),
            pl.BlockSpec((d, nh), lambda t: (0, 0)),
            pl.BlockSpec((d, nh), lambda t: (0, 0)),
            pl.BlockSpec((1, nh), lambda t: (0, 0)),
            pl.BlockSpec((1, nh), lambda t: (0, 0)),
            pl.BlockSpec((bm // 2, n), _aidx),
            pl.BlockSpec((bm // 2, n), _bidx),
            pl.BlockSpec((1, nh), lambda t: (0, 0)),
            pl.BlockSpec((1, nh), lambda t: (0, 0)),
            pl.BlockSpec((1, n), lambda t: (0, 0)),
            pl.BlockSpec((nh, nh), lambda t: (0, 0)),
        ],
        out_specs=[
            pl.BlockSpec((1, n), lambda t: (0, 0)),
            pl.BlockSpec((1, n), lambda t: (0, 0)),
        ],
        out_shape=[
            jax.ShapeDtypeStruct((1, n), jnp.float32),
            jax.ShapeDtypeStruct((1, n), jnp.float32),
        ],
        scratch_shapes=[
            pltpu.VMEM((n, 4 * nh), jnp.bfloat16),
            pltpu.VMEM((n, nh), jnp.bfloat16),
            pltpu.VMEM((n, nh), jnp.bfloat16),
        ],
        compiler_params=pltpu.CompilerParams(
            dimension_semantics=("arbitrary",)),
    )(s1, s2, W1, W2, b1.reshape(1, nh), b2.reshape(1, nh),
      A, A, av1, av2, msk, Wd)

    return jnp.concatenate(
        [o1 + samp_bias1 + bd[0], o2 + samp_bias2 + bd[0]], axis=1)
